# baseline (speedup 1.0000x reference)
import jax
import jax.numpy as jnp
from jax import lax
from jax.experimental import pallas as pl
from jax.experimental.pallas import tpu as pltpu
from jax.experimental.pallas import tpu_sc as plsc

B, H, S, D, Q = 16, 16, 2048, 128, 16
BH = B * H
ROWS = BH * S

CB = 8


def _tc_body(pos_ref, c_ref, val_ref, out_ref):
    out_ref[...] = c_ref[...]
    for c in range(CB):
        for q in range(Q):
            p = pos_ref[q]
            out_ref[c, pl.ds(p, 1), :] = val_ref[c, pl.ds(q, 1), :]


def _tc_copy_scatter(cache, val, input_pos):
    grid_spec = pltpu.PrefetchScalarGridSpec(
        num_scalar_prefetch=1,
        grid=(BH // CB,),
        in_specs=[
            pl.BlockSpec((CB, S, D), lambda i, pos: (i, 0, 0)),
            pl.BlockSpec((CB, Q, D), lambda i, pos: (i, 0, 0)),
        ],
        out_specs=pl.BlockSpec((CB, S, D), lambda i, pos: (i, 0, 0)),
    )
    return pl.pallas_call(
        _tc_body,
        grid_spec=grid_spec,
        out_shape=jax.ShapeDtypeStruct((BH, S, D), jnp.float32),
        compiler_params=pltpu.CompilerParams(
            dimension_semantics=("arbitrary",),
        ),
    )(input_pos, cache, val)


NC, NS = 2, 16
NW = NC * NS
SC_BH = 160
SC_ROWS = SC_BH * S
RW = SC_ROWS // NW
R = 256
NCH = RW // R
NBUF = 2
QW = (SC_BH * Q) // NW


def _sc_body(vc_ref, vv_ref, idx_ref, vo_ref,
             buf, idx_v, val_v, insem, outsem):
    cid = lax.axis_index("c")
    sid = lax.axis_index("s")
    wid = sid * NC + cid
    base = wid * RW

    for b in range(NBUF):
        pltpu.make_async_copy(vc_ref.at[pl.ds(base + b * R, R)],
                              buf.at[b], insem.at[b]).start()

    @pl.loop(0, NCH // NBUF)
    def _grp(g):
        for b in range(NBUF):
            c = g * NBUF + b
            off = base + c * R
            pltpu.make_async_copy(vc_ref.at[pl.ds(off, R)],
                                  buf.at[b], insem.at[b]).wait()
            pltpu.make_async_copy(buf.at[b], vo_ref.at[pl.ds(off, R)],
                                  outsem.at[b]).start()
            nxt = c + NBUF

            @pl.when(nxt < NCH)
            def _():
                pltpu.make_async_copy(buf.at[b], vo_ref.at[pl.ds(off, R)],
                                      outsem.at[b]).wait()
                pltpu.make_async_copy(vc_ref.at[pl.ds(base + nxt * R, R)],
                                      buf.at[b], insem.at[b]).start()

    for b in range(NBUF):
        pltpu.make_async_copy(buf.at[b], vo_ref.at[pl.ds(base, R)],
                              outsem.at[b]).wait()

    pltpu.sync_copy(idx_ref.at[pl.ds(wid, 1)], idx_v)
    pltpu.sync_copy(vv_ref.at[pl.ds(wid * QW, QW)], val_v)
    pltpu.sync_copy(val_v, vo_ref.at[idx_v.at[0]])


def _sc_partial_copy_scatter(cache_flat, val_flat, idx):
    mesh = plsc.VectorSubcoreMesh(core_axis_name="c", subcore_axis_name="s")
    kern = pl.kernel(
        _sc_body,
        out_type=jax.ShapeDtypeStruct((ROWS, D), jnp.float32),
        mesh=mesh,
        scratch_types=[
            pltpu.VMEM((NBUF, R, D), jnp.float32),
            pltpu.VMEM((1, QW), jnp.int32),
            pltpu.VMEM((QW, D), jnp.float32),
            pltpu.SemaphoreType.DMA((NBUF,)),
            pltpu.SemaphoreType.DMA((NBUF,)),
        ],
    )
    return kern(cache_flat, val_flat, idx)


TB = BH - SC_BH


def _tcb_body(part_ref, c_ref, val_ref, pos_ref, out_ref):
    out_ref[...] = c_ref[...]
    for c in range(CB):
        for q in range(Q):
            p = pos_ref[q]
            out_ref[pl.ds(c * S + p, 1), :] = val_ref[pl.ds(c * Q + q, 1), :]


def _tc_finish_v(v_partial, cache_flat, val_flat, input_pos):
    return pl.pallas_call(
        _tcb_body,
        grid=(TB // CB,),
        in_specs=[
            pl.BlockSpec(memory_space=pltpu.MemorySpace.HBM),
            pl.BlockSpec((CB * S, D), lambda i: (SC_BH // CB + i, 0)),
            pl.BlockSpec((CB * Q, D), lambda i: (SC_BH // CB + i, 0)),
            pl.BlockSpec(memory_space=pltpu.SMEM),
        ],
        out_specs=pl.BlockSpec((CB * S, D), lambda i: (SC_BH // CB + i, 0)),
        out_shape=jax.ShapeDtypeStruct((ROWS, D), jnp.float32),
        input_output_aliases={0: 0},
        compiler_params=pltpu.CompilerParams(
            dimension_semantics=("arbitrary",),
        ),
    )(v_partial, cache_flat, val_flat, input_pos)


def kernel(k_cache, v_cache, input_pos, k_val, v_val):
    vc_flat = v_cache.reshape(ROWS, D)
    vv_flat = v_val.reshape(BH * Q, D)

    idx = (jnp.arange(SC_BH, dtype=jnp.int32)[:, None] * S
           + input_pos[None, :].astype(jnp.int32)).reshape(NW, QW)
    v_part = _sc_partial_copy_scatter(vc_flat, vv_flat, idx)

    k_out = _tc_copy_scatter(k_cache.reshape(BH, S, D),
                             k_val.reshape(BH, Q, D), input_pos)

    v_out = _tc_finish_v(v_part, vc_flat, vv_flat, input_pos)

    return (k_out.reshape(B, H, S, D), v_out.reshape(B, H, S, D))


# speedup vs baseline: 1.0357x; 1.0357x over previous
import jax
import jax.numpy as jnp
from jax.experimental import pallas as pl
from jax.experimental.pallas import tpu as pltpu

B, H, S, D, Q = 16, 16, 2048, 128, 16
BH = B * H
CB = 8
NCHUNK = BH // CB
NITEM = 2 * NCHUNK
NBUF = 4


def _body(pos_ref, kc_ref, vc_ref, kv_ref, vv_ref, ko_ref, vo_ref,
          buf, insem, outsem):
    def item(j):
        c = j % 2
        i = j // 2
        src = kc_ref if c == 0 else vc_ref
        dst = ko_ref if c == 0 else vo_ref
        val = kv_ref if c == 0 else vv_ref
        return src, dst, val, i

    def start_in(j):
        b = j % NBUF
        src, dst, val, i = item(j)
        pltpu.make_async_copy(src.at[pl.ds(i * CB, CB)], buf.at[b],
                              insem.at[b]).start()

    def wait_in(j):
        b = j % NBUF
        src, dst, val, i = item(j)
        pltpu.make_async_copy(src.at[pl.ds(i * CB, CB)], buf.at[b],
                              insem.at[b]).wait()

    def start_out(j):
        b = j % NBUF
        src, dst, val, i = item(j)
        pltpu.make_async_copy(buf.at[b], dst.at[pl.ds(i * CB, CB)],
                              outsem.at[b]).start()

    def wait_out(j):
        b = j % NBUF
        src, dst, val, i = item(j)
        pltpu.make_async_copy(buf.at[b], dst.at[pl.ds(i * CB, CB)],
                              outsem.at[b]).wait()

    def scatter(j):
        b = j % NBUF
        src, dst, val, i = item(j)
        for c in range(CB):
            for q in range(Q):
                p = pos_ref[q]
                buf[b, c, pl.ds(p, 1), :] = val[i * CB + c, pl.ds(q, 1), :]

    for j in range(NBUF):
        start_in(j)
    for j in range(NITEM):
        wait_in(j)
        scatter(j)
        start_out(j)
        jn = j + NBUF
        if jn < NITEM:
            wait_out(j)
            start_in(jn)
    for j in range(NITEM - NBUF, NITEM):
        wait_out(j)


def kernel(k_cache, v_cache, input_pos, k_val, v_val):
    kc = k_cache.reshape(BH, S, D)
    vc = v_cache.reshape(BH, S, D)
    kv = k_val.reshape(BH, Q, D)
    vv = v_val.reshape(BH, Q, D)

    grid_spec = pltpu.PrefetchScalarGridSpec(
        num_scalar_prefetch=1,
        grid=(1,),
        in_specs=[
            pl.BlockSpec(memory_space=pltpu.MemorySpace.HBM),
            pl.BlockSpec(memory_space=pltpu.MemorySpace.HBM),
            pl.BlockSpec((BH, Q, D), lambda i, pos: (0, 0, 0)),
            pl.BlockSpec((BH, Q, D), lambda i, pos: (0, 0, 0)),
        ],
        out_specs=[
            pl.BlockSpec(memory_space=pltpu.MemorySpace.HBM),
            pl.BlockSpec(memory_space=pltpu.MemorySpace.HBM),
        ],
        scratch_shapes=[
            pltpu.VMEM((NBUF, CB, S, D), jnp.float32),
            pltpu.SemaphoreType.DMA((NBUF,)),
            pltpu.SemaphoreType.DMA((NBUF,)),
        ],
    )

    k_out, v_out = pl.pallas_call(
        _body,
        grid_spec=grid_spec,
        out_shape=[
            jax.ShapeDtypeStruct((BH, S, D), jnp.float32),
            jax.ShapeDtypeStruct((BH, S, D), jnp.float32),
        ],
    )(input_pos, kc, vc, kv, vv)

    return (k_out.reshape(B, H, S, D), v_out.reshape(B, H, S, D))
